# baseline (speedup 1.0000x reference)
import functools

import jax
import jax.numpy as jnp
from jax import lax
from jax.experimental import pallas as pl
from jax.experimental.pallas import tpu as pltpu
from jax.experimental.pallas import tpu_sc as plsc

_NEG = 200
_CI = 200
_D = 128
_GATHER_TILES = 13
_CSUM_BASE = 200
_PACK_ROWS = 216
_LAST_OFF = 184


@functools.cache
def _make_sc_gather():
    @functools.partial(
        pl.kernel,
        out_type=jax.ShapeDtypeStruct((_PACK_ROWS, _D), jnp.float32),
        mesh=plsc.VectorSubcoreMesh(core_axis_name="c", subcore_axis_name="s",
                                    num_cores=1),
        scratch_types=[
            pltpu.VMEM((32,), jnp.int32),
            pltpu.VMEM((32, _D), jnp.float32),
            pltpu.VMEM((1, _D), jnp.float32),
            pltpu.SemaphoreType.DMA,
            pltpu.SemaphoreType.DMA,
        ],
    )
    def _sc_gather(ci_hbm, neg_hbm, pp_hbm, out_hbm,
                   idx32, rows32, srow, sem1, sem2):
        s = lax.axis_index("s")

        @pl.when(s < _GATHER_TILES)
        def _():
            base = jnp.minimum(s * 16, _LAST_OFF)
            c2 = pltpu.async_copy(ci_hbm.at[pl.ds(base, 16)],
                                  idx32.at[pl.ds(16, 16)], sem2)
            c1 = pltpu.async_copy(neg_hbm.at[pl.ds(base, 16)],
                                  idx32.at[pl.ds(0, 16)], sem1)
            c2.wait()
            g2 = pltpu.async_copy(pp_hbm.at[idx32.at[pl.ds(16, 16)]],
                                  rows32.at[pl.ds(16, 16)], sem2)
            c1.wait()
            g1 = pltpu.async_copy(pp_hbm.at[idx32.at[pl.ds(0, 16)]],
                                  rows32.at[pl.ds(0, 16)], sem1)
            g2.wait()
            lo_w = jnp.broadcast_to(
                jnp.where(s < _GATHER_TILES - 1, 1.0, 0.0).astype(jnp.float32),
                (16,))
            for c in range(_D // 16):
                sl = pl.ds(c * 16, 16)
                lo = rows32[16, sl]
                for r in range(17, 24):
                    lo = lo + rows32[r, sl]
                hi = rows32[24, sl]
                for r in range(25, 32):
                    hi = hi + rows32[r, sl]
                srow[0, sl] = lo * lo_w + hi
            w2 = pltpu.async_copy(srow, out_hbm.at[pl.ds(_CSUM_BASE + s, 1)],
                                  sem2)
            g1.wait()
            w1 = pltpu.async_copy(rows32.at[pl.ds(0, 16)],
                                  out_hbm.at[pl.ds(base, 16)], sem1)
            w2.wait()
            w1.wait()

    return _sc_gather


def _finish_body(uid_ref, pid_ref, x_ref, up_ref, pp_ref, o_ref,
                 u_v, pi_v, sem1, sem2):
    cu = pltpu.make_async_copy(up_ref.at[pl.ds(uid_ref[0], 1)], u_v, sem1)
    cp = pltpu.make_async_copy(pp_ref.at[pl.ds(pid_ref[0], 1)], pi_v, sem2)
    cu.start()
    cp.start()

    neg = x_ref[0:_NEG, :]
    csum = jnp.sum(x_ref[_CSUM_BASE:_CSUM_BASE + _GATHER_TILES, :],
                   axis=0, keepdims=True)
    cu.wait()
    cp.wait()
    u = u_v[...]
    pi = pi_v[...]

    s = jnp.sum(u * pi)
    t = jnp.sum(csum * pi) / float(_CI)
    dn = (((1,), (1,)), ((), ()))
    a = lax.dot_general(u, neg, dn)
    b = lax.dot_general(csum, neg, dn) / float(_CI)

    score = jax.nn.sigmoid(s) * jax.nn.sigmoid(t)
    neg_score = jax.nn.sigmoid(a) * jax.nn.sigmoid(b)
    loss = -(jnp.log(score) + jnp.sum(jnp.log(1.0 - neg_score)))
    o_ref[0, 0] = loss


_finish = pl.pallas_call(
    _finish_body,
    in_specs=[
        pl.BlockSpec(memory_space=pltpu.SMEM),
        pl.BlockSpec(memory_space=pltpu.SMEM),
        pl.BlockSpec(memory_space=pltpu.VMEM),
        pl.BlockSpec(memory_space=pl.ANY),
        pl.BlockSpec(memory_space=pl.ANY),
    ],
    out_specs=pl.BlockSpec(memory_space=pltpu.SMEM),
    out_shape=jax.ShapeDtypeStruct((1, 1), jnp.float32),
    scratch_shapes=[
        pltpu.VMEM((1, _D), jnp.float32),
        pltpu.VMEM((1, _D), jnp.float32),
        pltpu.SemaphoreType.DMA,
        pltpu.SemaphoreType.DMA,
    ],
)


def kernel(userid, poii, Ci, neg_p, UserPreference, PoiPreference):
    packed = _make_sc_gather()(
        Ci.astype(jnp.int32), neg_p.astype(jnp.int32), PoiPreference)
    return _finish(userid.astype(jnp.int32), poii.astype(jnp.int32),
                   packed, UserPreference, PoiPreference)[0, 0]


# speedup vs baseline: 1.0373x; 1.0373x over previous
import functools

import jax
import jax.numpy as jnp
from jax import lax
from jax.experimental import pallas as pl
from jax.experimental.pallas import tpu as pltpu
from jax.experimental.pallas import tpu_sc as plsc

_NEG = 200
_CI = 200
_D = 128
_GATHER_TILES = 13
_PI_TILE = 13
_U_TILE = 14
_PI_ROW = 200
_CSUM_BASE = 201
_U_ROW = 214
_PACK_ROWS = 216
_LAST_OFF = 184


@functools.cache
def _make_sc_gather():
    @functools.partial(
        pl.kernel,
        out_type=jax.ShapeDtypeStruct((_PACK_ROWS, _D), jnp.float32),
        mesh=plsc.VectorSubcoreMesh(core_axis_name="c", subcore_axis_name="s",
                                    num_cores=1),
        scratch_types=[
            pltpu.VMEM((32,), jnp.int32),
            pltpu.VMEM((32, _D), jnp.float32),
            pltpu.VMEM((1, _D), jnp.float32),
            pltpu.SemaphoreType.DMA,
            pltpu.SemaphoreType.DMA,
        ],
    )
    def _sc_gather(uid_hbm, pid_hbm, ci_hbm, neg_hbm, up_hbm, pp_hbm, out_hbm,
                   idx32, rows32, srow, sem1, sem2):
        s = lax.axis_index("s")

        @pl.when(s < _GATHER_TILES)
        def _():
            base = jnp.minimum(s * 16, _LAST_OFF)
            c1 = pltpu.async_copy(neg_hbm.at[pl.ds(base, 16)],
                                  idx32.at[pl.ds(0, 16)], sem1)
            c2 = pltpu.async_copy(ci_hbm.at[pl.ds(base, 16)],
                                  idx32.at[pl.ds(16, 16)], sem2)
            c1.wait()
            g1 = pltpu.async_copy(pp_hbm.at[idx32.at[pl.ds(0, 16)]],
                                  rows32.at[pl.ds(0, 16)], sem1)
            c2.wait()
            g2 = pltpu.async_copy(pp_hbm.at[idx32.at[pl.ds(16, 16)]],
                                  rows32.at[pl.ds(16, 16)], sem2)
            g1.wait()
            w1 = pltpu.async_copy(rows32.at[pl.ds(0, 16)],
                                  out_hbm.at[pl.ds(base, 16)], sem1)
            g2.wait()
            lo_w = jnp.broadcast_to(
                jnp.where(s < _GATHER_TILES - 1, 1.0, 0.0).astype(jnp.float32),
                (16,))
            for c in range(_D // 16):
                sl = pl.ds(c * 16, 16)
                lo = rows32[16, sl]
                for r in range(17, 24):
                    lo = lo + rows32[r, sl]
                hi = rows32[24, sl]
                for r in range(25, 32):
                    hi = hi + rows32[r, sl]
                srow[0, sl] = lo * lo_w + hi
            w2 = pltpu.async_copy(srow, out_hbm.at[pl.ds(_CSUM_BASE + s, 1)],
                                  sem2)
            w1.wait()
            w2.wait()

        @pl.when(s == _PI_TILE)
        def _():
            pltpu.sync_copy(pid_hbm, idx32.at[pl.ds(0, 1)])
            i = idx32[...][0]
            pltpu.sync_copy(pp_hbm.at[pl.ds(i, 1)], out_hbm.at[pl.ds(_PI_ROW, 1)])

        @pl.when(s == _U_TILE)
        def _():
            pltpu.sync_copy(uid_hbm, idx32.at[pl.ds(0, 1)])
            i = idx32[...][0]
            pltpu.sync_copy(up_hbm.at[pl.ds(i, 1)], out_hbm.at[pl.ds(_U_ROW, 1)])

    return _sc_gather


def _finish_body(x_ref, o_ref):
    neg = x_ref[0:_NEG, :]
    pi = x_ref[_PI_ROW:_PI_ROW + 1, :]
    u = x_ref[_U_ROW:_U_ROW + 1, :]
    csum = jnp.sum(x_ref[_CSUM_BASE:_CSUM_BASE + _GATHER_TILES, :],
                   axis=0, keepdims=True)

    s = jnp.sum(u * pi)
    t = jnp.sum(csum * pi) / float(_CI)
    dn = (((1,), (1,)), ((), ()))
    a = lax.dot_general(u, neg, dn)
    b = lax.dot_general(csum, neg, dn) / float(_CI)

    score = jax.nn.sigmoid(s) * jax.nn.sigmoid(t)
    neg_score = jax.nn.sigmoid(a) * jax.nn.sigmoid(b)
    loss = -(jnp.log(score) + jnp.sum(jnp.log(1.0 - neg_score)))
    o_ref[0, 0] = loss


_finish = pl.pallas_call(
    _finish_body,
    out_shape=jax.ShapeDtypeStruct((1, 1), jnp.float32),
    out_specs=pl.BlockSpec(memory_space=pltpu.SMEM),
)


def kernel(userid, poii, Ci, neg_p, UserPreference, PoiPreference):
    packed = _make_sc_gather()(
        userid.astype(jnp.int32), poii.astype(jnp.int32),
        Ci.astype(jnp.int32), neg_p.astype(jnp.int32),
        UserPreference, PoiPreference)
    return _finish(packed)[0, 0]
